# Initial kernel scaffold; baseline (speedup 1.0000x reference)
import jax
import jax.numpy as jnp
from jax.experimental import pallas as pl


def kernel(hv, edge_index, he, Wm0, bm0, Wih0, Whh0, bih0, bhh0, Wm1, bm1, Wih1, Whh1, bih1, bhh1):
    raise NotImplementedError("write your pallas kernel here")


# speedup vs baseline: 10.8455x; 10.8455x over previous
import functools
import jax
import jax.numpy as jnp
from jax import lax
from jax.experimental import pallas as pl
from jax.experimental.pallas import tpu as pltpu, tpu_sc as plsc

N = 10000
E = 320000
D = 128
ACT = 2 * D

NC = 2
NS = 16
NW = NC * NS
HD = D // 2
EPT = E // NS
CH = 125
NCH = EPT // CH
EPW = E // NW
NPAD = 10240
RPT = NPAD // NS

CHA = 128
NCHA = EPW // CHA
AUXMAIN = NCHA * CHA
AUXTAIL = EPW - AUXMAIN

_mesh = plsc.VectorSubcoreMesh(
    core_axis_name="c", subcore_axis_name="s", num_cores=NC, num_subcores=NS)
_sc_params = pltpu.CompilerParams(use_tc_tiling_on_sc=False)


@functools.partial(
    pl.kernel,
    out_type=jax.ShapeDtypeStruct((NC, NPAD, HD), jnp.float32),
    mesh=_mesh,
    compiler_params=_sc_params,
    scratch_types=[
        pltpu.VMEM((NCH, CH), jnp.int32),
        pltpu.VMEM((NCH, CH), jnp.int32),
        pltpu.VMEM((CH, HD), jnp.float32),
        pltpu.VMEM((CH, HD), jnp.float32),
        pltpu.VMEM_SHARED((NPAD, HD), jnp.float32),
        pltpu.SemaphoreType.DMA,
        pltpu.SemaphoreType.DMA,
    ],
)
def _sc_segsum(hsplit_hbm, srcm_hbm, dstm_hbm, zrows_hbm, out_hbm,
               srcv, dstv, rows0, rows1, acc, sem0, sem1):
    cid = lax.axis_index("c")
    sid = lax.axis_index("s")
    h_hbm = hsplit_hbm.at[cid]

    pltpu.sync_copy(zrows_hbm, acc.at[pl.ds(sid * RPT, RPT)])
    pltpu.sync_copy(srcm_hbm.at[sid], srcv)
    pltpu.sync_copy(dstm_hbm.at[sid], dstv)
    plsc.subcore_barrier()

    pltpu.async_copy(h_hbm.at[srcv.at[0]], rows0, sem0)

    def pair_body(k, carry):
        j0 = 2 * k
        j1 = j0 + 1
        pltpu.make_async_copy(h_hbm.at[srcv.at[j0]], rows0, sem0).wait()
        pltpu.async_copy(h_hbm.at[srcv.at[j1]], rows1, sem1)
        pltpu.sync_copy(rows0, acc.at[dstv.at[j0]], add=True)
        pltpu.make_async_copy(h_hbm.at[srcv.at[j1]], rows1, sem1).wait()

        @pl.when(j1 + 1 < NCH)
        def _():
            pltpu.async_copy(h_hbm.at[srcv.at[j1 + 1]], rows0, sem0)

        pltpu.sync_copy(rows1, acc.at[dstv.at[j1]], add=True)
        return carry

    lax.fori_loop(0, NCH // 2, pair_body, 0)

    plsc.subcore_barrier()
    pltpu.sync_copy(acc.at[pl.ds(sid * RPT, RPT)],
                    out_hbm.at[cid, pl.ds(sid * RPT, RPT)])


@functools.partial(
    pl.kernel,
    out_type=jax.ShapeDtypeStruct((NC, NPAD, 16), jnp.float32),
    mesh=_mesh,
    compiler_params=_sc_params,
    scratch_types=[
        pltpu.VMEM((NCHA, CHA), jnp.int32),
        pltpu.VMEM((AUXTAIL,), jnp.int32),
        pltpu.VMEM((CHA, 16), jnp.float32),
        pltpu.VMEM((CHA, 16), jnp.float32),
        pltpu.VMEM_SHARED((NPAD, 16), jnp.float32),
        pltpu.SemaphoreType.DMA,
        pltpu.SemaphoreType.DMA,
    ],
)
def _sc_aux(erows_hbm, dstm_hbm, dstt_hbm, z16_hbm, out_hbm,
            dstv, dtv, arows0, arows1, acc, sem0, sem1):
    cid = lax.axis_index("c")
    sid = lax.axis_index("s")
    wid = cid * NS + sid

    pltpu.sync_copy(z16_hbm, acc.at[pl.ds(sid * RPT, RPT)])
    pltpu.sync_copy(dstm_hbm.at[wid], dstv)
    pltpu.sync_copy(dstt_hbm.at[wid], dtv)
    plsc.subcore_barrier()

    pltpu.async_copy(erows_hbm.at[wid, pl.ds(0, CHA)], arows0, sem0)

    def pair_body(k, carry):
        j0 = 2 * k
        j1 = j0 + 1
        pltpu.make_async_copy(erows_hbm.at[wid, pl.ds(j0 * CHA, CHA)],
                              arows0, sem0).wait()
        pltpu.async_copy(erows_hbm.at[wid, pl.ds(j1 * CHA, CHA)], arows1, sem1)
        pltpu.sync_copy(arows0, acc.at[dstv.at[j0]], add=True)
        pltpu.make_async_copy(erows_hbm.at[wid, pl.ds(j1 * CHA, CHA)],
                              arows1, sem1).wait()

        @pl.when(j1 + 1 < NCHA)
        def _():
            pltpu.async_copy(erows_hbm.at[wid, pl.ds((j1 + 1) * CHA, CHA)],
                             arows0, sem0)

        pltpu.sync_copy(arows1, acc.at[dstv.at[j1]], add=True)
        return carry

    lax.fori_loop(0, NCHA // 2, pair_body, 0)

    pltpu.sync_copy(erows_hbm.at[wid, pl.ds(AUXMAIN, AUXTAIL)],
                    arows0.at[pl.ds(0, AUXTAIL)])
    pltpu.sync_copy(arows0.at[pl.ds(0, AUXTAIL)], acc.at[dtv[...]], add=True)

    plsc.subcore_barrier()
    pltpu.sync_copy(acc.at[pl.ds(sid * RPT, RPT)],
                    out_hbm.at[cid, pl.ds(sid * RPT, RPT)])


BN = 1000

def _tc_round_body(h_ref, s_ref, a_ref,
                   A2_ref, B2_ref, cvec_ref, bm_ref,
                   WihT_ref, WhhT_ref, bih_ref, bhh_ref, out_ref):
    h = h_ref[...]
    aux = a_ref[0] + a_ref[1]
    she = aux[:, 0:1]
    deg = aux[:, 1:2]
    f32 = jnp.float32
    hp = jax.lax.Precision.HIGHEST
    hb = h.astype(jnp.bfloat16)
    a = (deg * (jnp.dot(hb, A2_ref[...], preferred_element_type=f32) + bm_ref[...])
         + jnp.dot(s_ref[0], B2_ref[:HD], preferred_element_type=f32, precision=hp)
         + jnp.dot(s_ref[1], B2_ref[HD:], preferred_element_type=f32, precision=hp)
         + she * cvec_ref[...])
    gi = jnp.dot(a.astype(jnp.bfloat16), WihT_ref[...], preferred_element_type=f32) + bih_ref[...]
    gh = jnp.dot(hb, WhhT_ref[...], preferred_element_type=f32) + bhh_ref[...]
    r = jax.nn.sigmoid(gi[:, :D] + gh[:, :D])
    z = jax.nn.sigmoid(gi[:, D:2 * D] + gh[:, D:2 * D])
    n = jnp.tanh(gi[:, 2 * D:] + r * gh[:, 2 * D:])
    out_ref[...] = (1.0 - z) * n + z * h


def _tc_round(h, S, aux, A2, B2, cvec, bm2, WihT, WhhT, bih2, bhh2):
    row_spec = lambda width: pl.BlockSpec((BN, width), lambda i: (i, 0))
    pair_spec = lambda width: pl.BlockSpec((NC, BN, width), lambda i: (0, i, 0))
    full = lambda shape: pl.BlockSpec(shape, lambda i: (0, 0))
    return pl.pallas_call(
        _tc_round_body,
        grid=(N // BN,),
        in_specs=[
            row_spec(D), pair_spec(HD), pair_spec(16),
            full((D, ACT)), full((D, ACT)), full((1, ACT)), full((1, ACT)),
            full((ACT, 3 * D)), full((D, 3 * D)), full((1, 3 * D)), full((1, 3 * D)),
        ],
        out_specs=row_spec(D),
        out_shape=jax.ShapeDtypeStruct((N, D), jnp.float32),
    )(h, S, aux, A2, B2, cvec, bm2, WihT, WhhT, bih2, bhh2)


def kernel(hv, edge_index, he,
           Wm0, bm0, Wih0, Whh0, bih0, bhh0,
           Wm1, bm1, Wih1, Whh1, bih1, bhh1):
    src = edge_index[0]
    dst = edge_index[1]
    srcm = src.reshape(NS, NCH, CH)
    dstm = dst.reshape(NS, NCH, CH)
    dst_w = dst.reshape(NW, EPW)
    dsta = dst_w[:, :AUXMAIN].reshape(NW, NCHA, CHA)
    dstt = dst_w[:, AUXMAIN:]
    he_r = he.astype(jnp.bfloat16).astype(jnp.float32)
    erows = jnp.concatenate(
        [he_r, jnp.ones((E, 1), jnp.float32), jnp.zeros((E, 14), jnp.float32)],
        axis=1).reshape(NW, EPW, 16)
    zrows = jnp.zeros((RPT, HD), jnp.float32)
    z16 = jnp.zeros((RPT, 16), jnp.float32)

    aux = _sc_aux(erows, dsta, dstt, z16)

    h = hv
    for (Wm, bm, Wih, Whh, bih, bhh) in (
            (Wm0, bm0, Wih0, Whh0, bih0, bhh0),
            (Wm1, bm1, Wih1, Whh1, bih1, bhh1)):
        bf16, f32 = jnp.bfloat16, jnp.float32
        WmT = Wm.T
        A2 = WmT[:D].astype(bf16)
        B2 = WmT[D:2 * D].astype(bf16).astype(f32)
        cvec = WmT[2 * D:2 * D + 1].astype(bf16).astype(f32)
        h_r = h.astype(bf16).astype(f32)
        h_split = jnp.stack([h_r[:, :HD], h_r[:, HD:]])
        S = _sc_segsum(h_split, srcm, dstm, zrows)
        h = _tc_round(h, S, aux,
                      A2, B2, cvec, bm.reshape(1, ACT),
                      Wih.T.astype(bf16), Whh.T.astype(bf16),
                      bih.reshape(1, 3 * D), bhh.reshape(1, 3 * D))
    return h


# speedup vs baseline: 10.9697x; 1.0115x over previous
import functools
import jax
import jax.numpy as jnp
from jax import lax
from jax.experimental import pallas as pl
from jax.experimental.pallas import tpu as pltpu, tpu_sc as plsc

N = 10000
E = 320000
D = 128
ACT = 2 * D

NC = 2
NS = 16
NW = NC * NS
HD = D // 2
EPT = E // NS
CH = 125
NCH = EPT // CH
EPW = E // NW
NPAD = 10240
RPT = NPAD // NS

CHA = 128
NCHA = EPW // CHA
AUXMAIN = NCHA * CHA
AUXTAIL = EPW - AUXMAIN

_mesh = plsc.VectorSubcoreMesh(
    core_axis_name="c", subcore_axis_name="s", num_cores=NC, num_subcores=NS)
_sc_params = pltpu.CompilerParams(use_tc_tiling_on_sc=False)


@functools.partial(
    pl.kernel,
    out_type=jax.ShapeDtypeStruct((NC, NPAD, HD), jnp.float32),
    mesh=_mesh,
    compiler_params=_sc_params,
    scratch_types=[
        pltpu.VMEM((NCH, CH), jnp.int32),
        pltpu.VMEM((NCH, CH), jnp.int32),
        pltpu.VMEM((CH, HD), jnp.float32),
        pltpu.VMEM((CH, HD), jnp.float32),
        pltpu.VMEM_SHARED((NPAD, HD), jnp.float32),
        pltpu.SemaphoreType.DMA,
        pltpu.SemaphoreType.DMA,
    ],
)
def _sc_segsum(hsplit_hbm, srcm_hbm, dstm_hbm, zrows_hbm, out_hbm,
               srcv, dstv, rows0, rows1, acc, sem0, sem1):
    cid = lax.axis_index("c")
    sid = lax.axis_index("s")
    h_hbm = hsplit_hbm.at[cid]

    pltpu.sync_copy(zrows_hbm, acc.at[pl.ds(sid * RPT, RPT)])
    pltpu.sync_copy(srcm_hbm.at[sid], srcv)
    pltpu.sync_copy(dstm_hbm.at[sid], dstv)
    plsc.subcore_barrier()

    pltpu.async_copy(h_hbm.at[srcv.at[0]], rows0, sem0)

    def pair_body(k, carry):
        j0 = 2 * k
        j1 = j0 + 1
        pltpu.make_async_copy(h_hbm.at[srcv.at[j0]], rows0, sem0).wait()
        pltpu.async_copy(h_hbm.at[srcv.at[j1]], rows1, sem1)
        pltpu.sync_copy(rows0, acc.at[dstv.at[j0]], add=True)
        pltpu.make_async_copy(h_hbm.at[srcv.at[j1]], rows1, sem1).wait()

        @pl.when(j1 + 1 < NCH)
        def _():
            pltpu.async_copy(h_hbm.at[srcv.at[j1 + 1]], rows0, sem0)

        pltpu.sync_copy(rows1, acc.at[dstv.at[j1]], add=True)
        return carry

    lax.fori_loop(0, NCH // 2, pair_body, 0)

    plsc.subcore_barrier()
    pltpu.sync_copy(acc.at[pl.ds(sid * RPT, RPT)],
                    out_hbm.at[cid, pl.ds(sid * RPT, RPT)])


@functools.partial(
    pl.kernel,
    out_type=jax.ShapeDtypeStruct((NC, NPAD, 16), jnp.float32),
    mesh=_mesh,
    compiler_params=_sc_params,
    scratch_types=[
        pltpu.VMEM((NCHA, CHA), jnp.int32),
        pltpu.VMEM((AUXTAIL,), jnp.int32),
        pltpu.VMEM((CHA, 16), jnp.float32),
        pltpu.VMEM((CHA, 16), jnp.float32),
        pltpu.VMEM_SHARED((NPAD, 16), jnp.float32),
        pltpu.SemaphoreType.DMA,
        pltpu.SemaphoreType.DMA,
    ],
)
def _sc_aux(erows_hbm, dstm_hbm, dstt_hbm, z16_hbm, out_hbm,
            dstv, dtv, arows0, arows1, acc, sem0, sem1):
    cid = lax.axis_index("c")
    sid = lax.axis_index("s")
    wid = cid * NS + sid

    pltpu.sync_copy(z16_hbm, acc.at[pl.ds(sid * RPT, RPT)])
    pltpu.sync_copy(dstm_hbm.at[wid], dstv)
    pltpu.sync_copy(dstt_hbm.at[wid], dtv)
    plsc.subcore_barrier()

    pltpu.async_copy(erows_hbm.at[wid, pl.ds(0, CHA)], arows0, sem0)

    def pair_body(k, carry):
        j0 = 2 * k
        j1 = j0 + 1
        pltpu.make_async_copy(erows_hbm.at[wid, pl.ds(j0 * CHA, CHA)],
                              arows0, sem0).wait()
        pltpu.async_copy(erows_hbm.at[wid, pl.ds(j1 * CHA, CHA)], arows1, sem1)
        pltpu.sync_copy(arows0, acc.at[dstv.at[j0]], add=True)
        pltpu.make_async_copy(erows_hbm.at[wid, pl.ds(j1 * CHA, CHA)],
                              arows1, sem1).wait()

        @pl.when(j1 + 1 < NCHA)
        def _():
            pltpu.async_copy(erows_hbm.at[wid, pl.ds((j1 + 1) * CHA, CHA)],
                             arows0, sem0)

        pltpu.sync_copy(arows1, acc.at[dstv.at[j1]], add=True)
        return carry

    lax.fori_loop(0, NCHA // 2, pair_body, 0)

    pltpu.sync_copy(erows_hbm.at[wid, pl.ds(AUXMAIN, AUXTAIL)],
                    arows0.at[pl.ds(0, AUXTAIL)])
    pltpu.sync_copy(arows0.at[pl.ds(0, AUXTAIL)], acc.at[dtv[...]], add=True)

    plsc.subcore_barrier()
    pltpu.sync_copy(acc.at[pl.ds(sid * RPT, RPT)],
                    out_hbm.at[cid, pl.ds(sid * RPT, RPT)])


BN = 1000

def _tc_round_body(h_ref, s_ref, a_ref,
                   A2_ref, B2_ref, cvec_ref, bm_ref,
                   WihT_ref, WhhT_ref, bih_ref, bhh_ref, out_ref, hs_ref):
    h = h_ref[...]
    aux = a_ref[0] + a_ref[1]
    she = aux[:, 0:1]
    deg = aux[:, 1:2]
    f32 = jnp.float32
    hp = jax.lax.Precision.HIGHEST
    hb = h.astype(jnp.bfloat16)
    a = (deg * (jnp.dot(hb, A2_ref[...], preferred_element_type=f32) + bm_ref[...])
         + jnp.dot(s_ref[0], B2_ref[:HD], preferred_element_type=f32, precision=hp)
         + jnp.dot(s_ref[1], B2_ref[HD:], preferred_element_type=f32, precision=hp)
         + she * cvec_ref[...])
    gi = jnp.dot(a.astype(jnp.bfloat16), WihT_ref[...], preferred_element_type=f32) + bih_ref[...]
    gh = jnp.dot(hb, WhhT_ref[...], preferred_element_type=f32) + bhh_ref[...]
    r = jax.nn.sigmoid(gi[:, :D] + gh[:, :D])
    z = jax.nn.sigmoid(gi[:, D:2 * D] + gh[:, D:2 * D])
    n = jnp.tanh(gi[:, 2 * D:] + r * gh[:, 2 * D:])
    hn = (1.0 - z) * n + z * h
    out_ref[...] = hn
    hr = hn.astype(jnp.bfloat16).astype(f32)
    hs_ref[0] = hr[:, :HD]
    hs_ref[1] = hr[:, HD:]


def _tc_round(h, S, aux, A2, B2, cvec, bm2, WihT, WhhT, bih2, bhh2):
    row_spec = lambda width: pl.BlockSpec((BN, width), lambda i: (i, 0))
    pair_spec = lambda width: pl.BlockSpec((NC, BN, width), lambda i: (0, i, 0))
    full = lambda shape: pl.BlockSpec(shape, lambda i: (0, 0))
    return pl.pallas_call(
        _tc_round_body,
        grid=(N // BN,),
        in_specs=[
            row_spec(D), pair_spec(HD), pair_spec(16),
            full((D, ACT)), full((D, ACT)), full((1, ACT)), full((1, ACT)),
            full((ACT, 3 * D)), full((D, 3 * D)), full((1, 3 * D)), full((1, 3 * D)),
        ],
        out_specs=[row_spec(D), pair_spec(D // 2)],
        out_shape=[jax.ShapeDtypeStruct((N, D), jnp.float32),
                   jax.ShapeDtypeStruct((NC, N, HD), jnp.float32)],
    )(h, S, aux, A2, B2, cvec, bm2, WihT, WhhT, bih2, bhh2)


def kernel(hv, edge_index, he,
           Wm0, bm0, Wih0, Whh0, bih0, bhh0,
           Wm1, bm1, Wih1, Whh1, bih1, bhh1):
    src = edge_index[0]
    dst = edge_index[1]
    srcm = src.reshape(NS, NCH, CH)
    dstm = dst.reshape(NS, NCH, CH)
    dst_w = dst.reshape(NW, EPW)
    dsta = dst_w[:, :AUXMAIN].reshape(NW, NCHA, CHA)
    dstt = dst_w[:, AUXMAIN:]
    he_r = he.astype(jnp.bfloat16).astype(jnp.float32)
    erows = jnp.concatenate(
        [he_r, jnp.ones((E, 1), jnp.float32), jnp.zeros((E, 14), jnp.float32)],
        axis=1).reshape(NW, EPW, 16)
    zrows = jnp.zeros((RPT, HD), jnp.float32)
    z16 = jnp.zeros((RPT, 16), jnp.float32)

    aux = _sc_aux(erows, dsta, dstt, z16)

    h = hv
    h_r0 = hv.astype(jnp.bfloat16).astype(jnp.float32)
    h_split = jnp.stack([h_r0[:, :HD], h_r0[:, HD:]])
    for (Wm, bm, Wih, Whh, bih, bhh) in (
            (Wm0, bm0, Wih0, Whh0, bih0, bhh0),
            (Wm1, bm1, Wih1, Whh1, bih1, bhh1)):
        bf16, f32 = jnp.bfloat16, jnp.float32
        WmT = Wm.T
        A2 = WmT[:D].astype(bf16)
        B2 = WmT[D:2 * D].astype(bf16).astype(f32)
        cvec = WmT[2 * D:2 * D + 1].astype(bf16).astype(f32)
        S = _sc_segsum(h_split, srcm, dstm, zrows)
        h, h_split = _tc_round(h, S, aux,
                               A2, B2, cvec, bm.reshape(1, ACT),
                               Wih.T.astype(bf16), Whh.T.astype(bf16),
                               bih.reshape(1, 3 * D), bhh.reshape(1, 3 * D))
    return h
